# baseline (speedup 1.0000x reference)
import functools

import jax
import jax.numpy as jnp
import numpy as np
from jax import lax
from jax.experimental import pallas as pl
from jax.experimental.pallas import tpu as pltpu
from jax.experimental.pallas import tpu_sc as plsc

_B, _G = 16, 4096
_RATIO = 0.6
_LANES = 16
_CHUNKS = _G // _LANES
_WORDS = _G // (4 * _LANES)


def _rotl32(x, d):
    return ((x << np.uint32(d)) | (x >> np.uint32(32 - d))).astype(np.uint32)


def _threefry2x32(ks0, ks1, x0, x1):
    rotations = ((13, 15, 26, 6), (17, 29, 16, 24))
    ks = (np.uint32(ks0), np.uint32(ks1),
          np.uint32(ks0) ^ np.uint32(ks1) ^ np.uint32(0x1BD11BDA))
    x = [(x0 + ks[0]).astype(np.uint32), (x1 + ks[1]).astype(np.uint32)]
    for i in range(5):
        for r in rotations[i % 2]:
            x[0] = (x[0] + x[1]).astype(np.uint32)
            x[1] = _rotl32(x[1], r) ^ x[0]
        x[0] = (x[0] + ks[(i + 1) % 3]).astype(np.uint32)
        x[1] = (x[1] + ks[(i + 2) % 3] + np.uint32(i + 1)).astype(np.uint32)
    return x


def _uniform_scores():
    n = _B * _G
    hi = np.zeros(n, dtype=np.uint32)
    lo = np.arange(n, dtype=np.uint32)
    o0, o1 = _threefry2x32(0, 42, hi, lo)
    bits = o0 ^ o1
    f = ((bits >> np.uint32(9)) | np.uint32(0x3F800000)).view(np.float32)
    f = np.maximum(np.float32(0.0), f - np.float32(1.0))
    return f.reshape(_B, _G)


def _build_tables():
    scores = _uniform_scores()
    order = np.argsort(scores, axis=1, kind="stable").astype(np.int32)
    rank = np.empty_like(order)
    rank[np.arange(_B)[:, None], order] = np.broadcast_to(
        np.arange(_G, dtype=np.int32)[None, :], (_B, _G))
    rank_p = (rank.reshape(_B, _WORDS, _LANES, 4)
              .transpose(0, 1, 3, 2)
              .reshape(_B, _G))
    return order, rank_p


_ORDER, _RANKP = _build_tables()

_MESH = plsc.VectorSubcoreMesh(core_axis_name="c", subcore_axis_name="s")


@functools.partial(
    pl.kernel,
    out_type=(jax.ShapeDtypeStruct((_B, _G // 4), jnp.int32),
              jax.ShapeDtypeStruct((_B, _G // 4), jnp.int32)),
    mesh=_MESH,
    scratch_types=[
        pltpu.VMEM((_LANES,), jnp.int32),
        pltpu.VMEM((_G,), jnp.int32),
        pltpu.VMEM((_G,), jnp.int32),
        pltpu.VMEM((_G // 4,), jnp.int32),
        pltpu.SemaphoreType.DMA,
    ],
    compiler_params=pltpu.CompilerParams(needs_layout_passes=False),
)
def _mask_program(len_hbm, order_hbm, rank_hbm, m_hbm, nm_hbm,
                  len_v, order_v, rank_v, out_v, sem):
    c = lax.axis_index("c")
    s = lax.axis_index("s")
    row = s

    rank_dma = pltpu.async_copy(rank_hbm.at[row], rank_v, sem)
    pltpu.sync_copy(len_hbm.at[row], len_v)
    pltpu.sync_copy(order_hbm.at[row], order_v)

    l_splat = len_v[...]
    nmask_splat = (l_splat.astype(jnp.float32)
                   * jnp.float32(_RATIO)).astype(jnp.int32)

    def phase1(j, carry):
        run, tau_acc = carry
        chunk = order_v[pl.ds(j * _LANES, _LANES)]
        v = chunk < l_splat
        cs = plsc.cumsum(jnp.where(v, 1, 0).astype(jnp.int32))
        cincl = run + cs
        tau_acc = tau_acc + jnp.where(cincl <= nmask_splat, 1, 0)
        run = run + plsc.all_reduce_population_count(v)
        return run, tau_acc

    zeros = jnp.zeros((_LANES,), jnp.int32)
    _, tau_acc = lax.fori_loop(0, _CHUNKS, phase1, (zeros, zeros), unroll=4)
    tau = jnp.full((_LANES,), jnp.sum(tau_acc), dtype=jnp.int32)

    rank_dma.wait()
    iota4 = lax.iota(jnp.int32, _LANES) * 4
    flip = c == 1

    def phase2(j, carry):
        base = j * (4 * _LANES)
        word = jnp.zeros((_LANES,), jnp.int32)
        for t in range(4):
            r = rank_v[pl.ds(base + t * _LANES, _LANES)]
            p = iota4 + (base + t)
            sel = (p < l_splat) & ((r < tau) ^ flip)
            word = word | (jnp.where(sel, 1, 0) << (8 * t))
        out_v[pl.ds(j * _LANES, _LANES)] = word
        return carry

    lax.fori_loop(0, _WORDS, phase2, 0, unroll=2)

    @pl.when(c == 0)
    def _():
        pltpu.sync_copy(out_v, m_hbm.at[row])

    @pl.when(c == 1)
    def _():
        pltpu.sync_copy(out_v, nm_hbm.at[row])


def kernel(centers, lengths):
    del centers
    len2d = jnp.broadcast_to(lengths[:, None], (_B, _LANES))
    m_w, nm_w = _mask_program(len2d, _ORDER, _RANKP)
    return m_w.view(jnp.bool_), nm_w.view(jnp.bool_)
